# baseline (speedup 1.0000x reference)
import functools

import jax
import jax.numpy as jnp
from jax import lax
from jax.experimental import pallas as pl
from jax.experimental.pallas import tpu as pltpu
from jax.experimental.pallas import tpu_sc as plsc

_NUM_USERS = 1000000
_EMBED_DIM = 64
_BATCH = 16384

_NC = 2
_NS = 16
_NW = _NC * _NS
_B_PER_W = _BATCH // _NW
_L = 16
_NG = _B_PER_W // _L

_mesh = plsc.VectorSubcoreMesh(core_axis_name="c", subcore_axis_name="s")


@functools.partial(
    pl.kernel,
    mesh=_mesh,
    out_type=jax.ShapeDtypeStruct((_BATCH, _EMBED_DIM), jnp.float32),
    scratch_types=[
        pltpu.VMEM((_B_PER_W,), jnp.int32),
        pltpu.VMEM((_B_PER_W, _EMBED_DIM), jnp.float32),
        pltpu.SemaphoreType.DMA,
    ],
)
def _gather_kernel(idx_hbm, table_hbm, out_hbm, idx_v, out_v, sem):
    wid = lax.axis_index("s") * _NC + lax.axis_index("c")
    base = wid * _B_PER_W

    pltpu.sync_copy(idx_hbm.at[pl.ds(base, _B_PER_W)], idx_v)

    @plsc.parallel_loop(0, _B_PER_W, step=_L, unroll=2)
    def _fire(i0):
        rvec = idx_v[pl.ds(i0, _L)]
        for l in range(_L):
            pltpu.async_copy(table_hbm.at[rvec[l]], out_v.at[i0 + l], sem)

    def drain(i, carry):
        pltpu.make_async_copy(table_hbm.at[0], out_v.at[0], sem).wait()
        return carry

    lax.fori_loop(0, _B_PER_W, drain, jnp.int32(0))

    pltpu.sync_copy(out_v, out_hbm.at[pl.ds(base, _B_PER_W)])


def kernel(user_indices, embedding_table):
    return _gather_kernel(user_indices.astype(jnp.int32), embedding_table)


# speedup vs baseline: 1.0032x; 1.0032x over previous
import functools

import jax
import jax.numpy as jnp
from jax import lax
from jax.experimental import pallas as pl
from jax.experimental.pallas import tpu as pltpu
from jax.experimental.pallas import tpu_sc as plsc

_NUM_USERS = 1000000
_EMBED_DIM = 64
_BATCH = 16384

_NC = 2
_NS = 16
_NW = _NC * _NS
_B_PER_W = _BATCH // _NW
_L = 16
_NG = _B_PER_W // _L

_mesh = plsc.VectorSubcoreMesh(core_axis_name="c", subcore_axis_name="s")


@functools.partial(
    pl.kernel,
    mesh=_mesh,
    out_type=jax.ShapeDtypeStruct((_BATCH, _EMBED_DIM), jnp.float32),
    scratch_types=[
        pltpu.VMEM((_B_PER_W,), jnp.int32),
        pltpu.VMEM((_B_PER_W, _EMBED_DIM), jnp.float32),
    ] + [pltpu.SemaphoreType.DMA] * 8,
)
def _gather_kernel(idx_hbm, table_hbm, out_hbm, idx_v, out_v, *sems):
    wid = lax.axis_index("s") * _NC + lax.axis_index("c")
    base = wid * _B_PER_W

    pltpu.sync_copy(idx_hbm.at[pl.ds(base, _B_PER_W)], idx_v)

    @plsc.parallel_loop(0, _B_PER_W, step=_L, unroll=2)
    def _fire(i0):
        rvec = idx_v[pl.ds(i0, _L)]
        for l in range(_L):
            pltpu.async_copy(table_hbm.at[rvec[l]], out_v.at[i0 + l],
                             sems[l % 8])

    def drain(i, carry):
        for s in range(8):
            pltpu.make_async_copy(table_hbm.at[0], out_v.at[0],
                                  sems[s]).wait()
        return carry

    lax.fori_loop(0, _B_PER_W // 8, drain, jnp.int32(0))

    pltpu.sync_copy(out_v, out_hbm.at[pl.ds(base, _B_PER_W)])


def kernel(user_indices, embedding_table):
    return _gather_kernel(user_indices.astype(jnp.int32), embedding_table)
